# baseline (speedup 1.0000x reference)
import functools

import jax
import jax.numpy as jnp
from jax import lax
from jax.experimental import pallas as pl
from jax.experimental.pallas import tpu as pltpu
from jax.experimental.pallas import tpu_sc as plsc

L = 16
NC, NS = 2, 16
NW = NC * NS


def _build_counts(B, D):
    rows_pw = B // NW
    nfull = D // L
    tail = D - nfull * L
    buf_n = rows_pw * D

    mesh = plsc.VectorSubcoreMesh(core_axis_name="c", subcore_axis_name="s")

    @functools.partial(
        pl.kernel,
        mesh=mesh,
        out_type=jax.ShapeDtypeStruct((B * D,), jnp.float32),
        compiler_params=pltpu.CompilerParams(needs_layout_passes=False),
        scratch_types=[
            pltpu.VMEM((buf_n,), jnp.float32),
            pltpu.VMEM((buf_n,), jnp.float32),
            pltpu.SemaphoreType.DMA,
        ],
    )
    def counts(in_hbm, c_hbm, buf, cbuf, sem):
        wid = lax.axis_index("s") * NC + lax.axis_index("c")
        base = wid * buf_n
        cp = pltpu.async_copy(in_hbm.at[pl.ds(base, buf_n)], buf, sem)
        zero = jnp.zeros((L,), jnp.float32)

        def zero_fn(z, _):
            for u in range(4):
                cbuf[pl.ds(z * 4 * L + u * L, L)] = zero
            return 0

        lax.fori_loop(0, buf_n // (4 * L), zero_fn, 0)
        cp.wait()
        lane = lax.iota(jnp.int32, L)
        ones = jnp.full((L,), 1.0, jnp.float32)

        def row_fn(r, _):
            roff = r * D

            def jbody(j, c):
                inv = buf[pl.ds(roff + j * L, L)]
                plsc.addupdate_scatter(
                    cbuf, [inv.astype(jnp.int32) + roff], ones
                )
                return c

            lax.fori_loop(0, nfull, jbody, 0)
            taddr = jnp.where(lane < tail, roff + nfull * L + lane, roff)
            inv = plsc.load_gather(buf, [taddr])
            plsc.addupdate_scatter(
                cbuf, [inv.astype(jnp.int32) + roff], ones, mask=lane < tail
            )
            return 0

        lax.fori_loop(0, rows_pw, row_fn, 0)
        pltpu.sync_copy(cbuf, c_hbm.at[pl.ds(base, buf_n)])

    return counts


def _build_combine(B, D, K, blk):
    def body(x_ref, c_ref, v_ref, w_ref, o_ref):
        v = v_ref[...]
        c = c_ref[...]
        s = jnp.dot(c, v, preferred_element_type=jnp.float32)
        wd = jnp.dot(x_ref[...], w_ref[...],
                     preferred_element_type=jnp.float32)
        rn2 = jnp.sum(v * v, axis=1, keepdims=True)
        q = jnp.dot(c, rn2, preferred_element_type=jnp.float32)
        inter = 0.5 * (jnp.sum(s * s, axis=1) - q[:, 0])
        o_ref[0, 0, :] = wd[:, 0] + inter

    grid = B // blk
    return pl.pallas_call(
        body,
        grid=(grid,),
        in_specs=[
            pl.BlockSpec((blk, D), lambda i: (i, 0)),
            pl.BlockSpec((blk, D), lambda i: (i, 0)),
            pl.BlockSpec((D, K), lambda i: (0, 0)),
            pl.BlockSpec((D, 1), lambda i: (0, 0)),
        ],
        out_specs=pl.BlockSpec((1, 1, blk), lambda i: (i, 0, 0)),
        out_shape=jax.ShapeDtypeStruct((grid, 1, blk), jnp.float32),
    )


def kernel(inputs, w, v):
    B, D = inputs.shape
    Dv, K = v.shape
    in_flat = inputs.reshape(-1)
    c_flat = _build_counts(B, D)(in_flat)
    counts2d = c_flat.reshape(B, D)
    out = _build_combine(B, D, K, 128)(inputs, counts2d, v, w)
    return out.reshape(B)
